# baseline (speedup 1.0000x reference)
import functools

import jax
import jax.numpy as jnp
from jax import lax
from jax.experimental import pallas as pl
from jax.experimental.pallas import tpu as pltpu
from jax.experimental.pallas import tpu_sc as plsc

VOCAB = 1000000
DIM = 64
B = 4096
S = 200

NC = 2
NS = 16
NW = NC * NS
SEQ_PER_W = B // NW
HALF = S // 2
LANES = 16


def _body(ids_hbm, embed_hbm, pos_hbm, out_hbm, idx_v, pos_v, rows_v, semg):
    wid = lax.axis_index("s") * NC + lax.axis_index("c")

    pltpu.sync_copy(ids_hbm.at[wid], idx_v)
    pltpu.sync_copy(pos_hbm, pos_v)

    def seq_body(n, carry):
        pltpu.async_copy(
            embed_hbm.at[idx_v.at[2 * n]], rows_v.at[pl.ds(0, HALF)], semg
        ).wait()
        pltpu.async_copy(
            embed_hbm.at[idx_v.at[2 * n + 1]], rows_v.at[pl.ds(HALF, HALF)], semg
        ).wait()

        def t_body(t, c):
            for l in range(DIM // LANES):
                sl = pl.ds(l * LANES, LANES)
                plsc.addupdate(rows_v.at[t, sl], pos_v[t, sl])
            return c

        lax.fori_loop(0, S, t_body, 0)
        pltpu.sync_copy(rows_v, out_hbm.at[wid, n])
        return carry

    lax.fori_loop(0, SEQ_PER_W, seq_body, 0)


@jax.jit
def _run(ids3, embed, pos_s):
    mesh = plsc.VectorSubcoreMesh(core_axis_name="c", subcore_axis_name="s")
    f = pl.kernel(
        _body,
        out_type=jax.ShapeDtypeStruct((NW, SEQ_PER_W, S, DIM), jnp.float32),
        mesh=mesh,
        scratch_types=[
            pltpu.VMEM((2 * SEQ_PER_W, HALF), jnp.int32),
            pltpu.VMEM((S, DIM), jnp.float32),
            pltpu.VMEM((S, DIM), jnp.float32),
            pltpu.SemaphoreType.DMA,
        ],
        compiler_params=pltpu.CompilerParams(use_tc_tiling_on_sc=False),
    )
    return f(ids3, embed, pos_s)


def kernel(input_ids, embed, pos):
    ids3 = input_ids.astype(jnp.int32).reshape(NW, 2 * SEQ_PER_W, HALF)
    pos_s = pos[0, :S]
    out = _run(ids3, embed, pos_s)
    return out.reshape(B, S, DIM)


# speedup vs baseline: 1.2225x; 1.2225x over previous
import jax
import jax.numpy as jnp
from jax import lax
from jax.experimental import pallas as pl
from jax.experimental.pallas import tpu as pltpu
from jax.experimental.pallas import tpu_sc as plsc

VOCAB = 1000000
DIM = 64
B = 4096
S = 200

NC = 2
NS = 16
NW = NC * NS
SEQ_PER_W = B // NW
HALF = S // 2
LANES = 16
NBUF = 4
LOOKAHEAD = 2


def _body(ids_hbm, embed_hbm, pos_hbm, out_hbm,
          idx_v, pos_v, rv0, rv1, rv2, rv3,
          sg0, sg1, sg2, sg3, so0, so1, so2, so3):
    rows = [rv0, rv1, rv2, rv3]
    semg = [sg0, sg1, sg2, sg3]
    semo = [so0, so1, so2, so3]
    wid = lax.axis_index("s") * NC + lax.axis_index("c")

    pltpu.sync_copy(ids_hbm.at[wid], idx_v)
    pltpu.sync_copy(pos_hbm, pos_v)

    def gather_desc(g, b):
        return (
            pltpu.make_async_copy(
                embed_hbm.at[idx_v.at[2 * g]],
                rows[b].at[pl.ds(0, HALF)], semg[b]),
            pltpu.make_async_copy(
                embed_hbm.at[idx_v.at[2 * g + 1]],
                rows[b].at[pl.ds(HALF, HALF)], semg[b]),
        )

    def out_desc(g, b):
        return pltpu.make_async_copy(rows[b], out_hbm.at[wid, g], semo[b])

    def fire_gather(g, b):
        d0, d1 = gather_desc(g, b)
        d0.start()
        d1.start()

    fire_gather(0, 0)
    fire_gather(1, 1)

    def group(j, carry):
        for b in range(NBUF):
            g = NBUF * j + b
            bn = (b + LOOKAHEAD) % NBUF

            @pl.when(g >= LOOKAHEAD)
            def _():
                out_desc(g - LOOKAHEAD, bn).wait()

            @pl.when(g + LOOKAHEAD < SEQ_PER_W)
            def _():
                fire_gather(g + LOOKAHEAD, bn)

            d0, d1 = gather_desc(g, b)
            d0.wait()
            d1.wait()

            def t_body(j2, c):
                for k in range(4):
                    t = 4 * j2 + k
                    for l in range(DIM // LANES):
                        sl = pl.ds(l * LANES, LANES)
                        plsc.addupdate(rows[b].at[t, sl], pos_v[t, sl])
                return c

            lax.fori_loop(0, S // 4, t_body, 0)
            out_desc(g, b).start()
        return carry

    lax.fori_loop(0, SEQ_PER_W // NBUF, group, 0)

    for g in range(SEQ_PER_W - LOOKAHEAD, SEQ_PER_W):
        out_desc(g, g % NBUF).wait()


@jax.jit
def _run(ids3, embed, pos_s):
    mesh = plsc.VectorSubcoreMesh(core_axis_name="c", subcore_axis_name="s")
    f = pl.kernel(
        _body,
        out_type=jax.ShapeDtypeStruct((NW, SEQ_PER_W, S, DIM), jnp.float32),
        mesh=mesh,
        scratch_types=[
            pltpu.VMEM((2 * SEQ_PER_W, HALF), jnp.int32),
            pltpu.VMEM((S, DIM), jnp.float32),
        ] + [pltpu.VMEM((S, DIM), jnp.float32)] * NBUF
          + [pltpu.SemaphoreType.DMA] * (2 * NBUF),
        compiler_params=pltpu.CompilerParams(use_tc_tiling_on_sc=False),
    )
    return f(ids3, embed, pos_s)


def kernel(input_ids, embed, pos):
    ids3 = input_ids.astype(jnp.int32).reshape(NW, 2 * SEQ_PER_W, HALF)
    pos_s = pos[0, :S]
    out = _run(ids3, embed, pos_s)
    return out.reshape(B, S, DIM)
